# baseline (speedup 1.0000x reference)
import jax
import jax.numpy as jnp
from jax.experimental import pallas as pl
from jax.experimental.pallas import tpu as pltpu


def _dm_kernel(doc_s, ctx_s, smp_s,
               doc_all,
               para_hbm,
               word3, outT3,
               o_ref,
               pbuf,
               psem):
    TB, S = o_ref.shape
    C = ctx_s.shape[1]
    i = pl.program_id(0)
    ngrid = pl.num_programs(0)

    U = 32 if TB % 32 == 0 else 8
    lane = jax.lax.broadcasted_iota(jnp.int32, (1, S), 1)

    cur = jax.lax.rem(i, 2)
    nxt = jax.lax.rem(i + 1, 2)

    def issue_tile(tile, slot):
        def issue(ci, carry):
            for u in range(U):
                k = ci * U + u
                pltpu.make_async_copy(
                    para_hbm.at[doc_all[tile * TB + k, 0]],
                    pbuf.at[slot * TB + k], psem.at[slot]).start()
            return carry
        jax.lax.fori_loop(0, TB // U, issue, 0)

    @pl.when(i == 0)
    def _():
        issue_tile(0, 0)

    @pl.when(i + 1 < ngrid)
    def _():
        issue_tile(i + 1, nxt)

    wait_view = pbuf.at[pl.ds(cur * TB, TB)]
    pltpu.make_async_copy(wait_view, wait_view, psem.at[cur]).wait()
    rofs = cur * TB

    def _tree_sum(vals):
        while len(vals) > 1:
            nxt = [a + b for a, b in zip(vals[::2], vals[1::2])]
            if len(vals) % 2:
                nxt.append(vals[-1])
            vals = nxt
        return vals[0]

    def chunk(ci, carry):
        rbase = ci * U
        rows = []
        for u in range(U):
            r = rbase + u
            terms = [pbuf[rofs + r]]
            for c in range(C):
                terms.append(word3[ctx_s[r, c]])
            acc = _tree_sum(terms)
            cols = []
            for s in range(S):
                g = outT3[smp_s[r, s]]
                v = jnp.sum(acc * g, axis=-1, keepdims=True)
                cols.append(jnp.where(lane == s, v, 0.0))
            rows.append(_tree_sum(cols))
        blk = jnp.concatenate(rows, axis=0)
        o_ref[pl.ds(pl.multiple_of(ci * U, U), U), :] = blk
        return carry

    jax.lax.fori_loop(0, TB // U, chunk, 0)


def kernel(doc_ids, context_ids, sample_ids, paragraph_matrix, word_matrix,
           outputs):
    B, C = context_ids.shape
    S = sample_ids.shape[1]
    n_docs, D = paragraph_matrix.shape
    n_words = word_matrix.shape[0]

    TB = 128 if B % 128 == 0 else 8
    B_pad = ((B + TB - 1) // TB) * TB

    pad_b = B_pad - B
    doc = doc_ids.astype(jnp.int32).reshape(B, 1)
    ctx = context_ids.astype(jnp.int32)
    smp = sample_ids.astype(jnp.int32)
    if pad_b:
        doc = jnp.pad(doc, ((0, pad_b), (0, 0)))
        ctx = jnp.pad(ctx, ((0, pad_b), (0, 0)))
        smp = jnp.pad(smp, ((0, pad_b), (0, 0)))

    para3 = paragraph_matrix.reshape(n_docs, 1, D)
    word3 = word_matrix.reshape(n_words, 1, D)
    outT3 = jnp.swapaxes(outputs, 0, 1).reshape(n_words, 1, D)

    res = pl.pallas_call(
        _dm_kernel,
        grid=(B_pad // TB,),
        in_specs=[
            pl.BlockSpec((TB, 1), lambda i: (i, 0),
                         memory_space=pltpu.SMEM),
            pl.BlockSpec((TB, C), lambda i: (i, 0),
                         memory_space=pltpu.SMEM),
            pl.BlockSpec((TB, S), lambda i: (i, 0),
                         memory_space=pltpu.SMEM),
            pl.BlockSpec((B_pad, 1), lambda i: (0, 0),
                         memory_space=pltpu.SMEM),
            pl.BlockSpec(memory_space=pl.ANY),
            pl.BlockSpec((n_words, 1, D), lambda i: (0, 0, 0)),
            pl.BlockSpec((n_words, 1, D), lambda i: (0, 0, 0)),
        ],
        out_specs=pl.BlockSpec((TB, S), lambda i: (i, 0)),
        out_shape=jax.ShapeDtypeStruct((B_pad, S), jnp.float32),
        scratch_shapes=[pltpu.VMEM((2 * TB, 1, D), jnp.float32),
                        pltpu.SemaphoreType.DMA((2,))],
        compiler_params=pltpu.CompilerParams(
            dimension_semantics=("arbitrary",),
            vmem_limit_bytes=64 * 1024 * 1024),
    )(doc, ctx, smp, doc, para3, word3, outT3)

    return jnp.squeeze(res[:B])


# speedup vs baseline: 7.1415x; 7.1415x over previous
import jax
import jax.numpy as jnp
from jax.experimental import pallas as pl
from jax.experimental.pallas import tpu as pltpu


def _dm_kernel(doc_s, ctx_s, smp_s,
               para_hbm,
               word3, outT3,
               o_ref,
               pbuf,
               psem):
    TB, S = o_ref.shape
    C = ctx_s.shape[1]

    U = 128 if TB % 128 == 0 else 8
    lane = jax.lax.broadcasted_iota(jnp.int32, (1, S), 1)

    def issue(ci, carry):
        for u in range(U):
            k = ci * U + u
            pltpu.make_async_copy(para_hbm.at[doc_s[k, 0]],
                                  pbuf.at[k], psem).start()
        return carry

    jax.lax.fori_loop(0, TB // U, issue, 0)
    pltpu.make_async_copy(pbuf, pbuf, psem).wait()

    def _tree_sum(vals):
        while len(vals) > 1:
            nxt = [a + b for a, b in zip(vals[::2], vals[1::2])]
            if len(vals) % 2:
                nxt.append(vals[-1])
            vals = nxt
        return vals[0]

    def chunk(ci, carry):
        rbase = ci * U
        rows = []
        for u in range(U):
            r = rbase + u
            terms = [pbuf[r]]
            for c in range(C):
                terms.append(word3[ctx_s[r, c]])
            acc = _tree_sum(terms)
            cols = []
            for s in range(S):
                g = outT3[smp_s[r, s]]
                v = jnp.sum(acc * g, axis=-1, keepdims=True)
                cols.append(jnp.where(lane == s, v, 0.0))
            rows.append(_tree_sum(cols))
        blk = jnp.concatenate(rows, axis=0)
        o_ref[pl.ds(pl.multiple_of(ci * U, U), U), :] = blk
        return carry

    jax.lax.fori_loop(0, TB // U, chunk, 0)


def kernel(doc_ids, context_ids, sample_ids, paragraph_matrix, word_matrix,
           outputs):
    B, C = context_ids.shape
    S = sample_ids.shape[1]
    n_docs, D = paragraph_matrix.shape
    n_words = word_matrix.shape[0]

    TB = 128 if B % 128 == 0 else 8
    B_pad = ((B + TB - 1) // TB) * TB

    pad_b = B_pad - B
    doc = doc_ids.astype(jnp.int32).reshape(B, 1)
    ctx = context_ids.astype(jnp.int32)
    smp = sample_ids.astype(jnp.int32)
    if pad_b:
        doc = jnp.pad(doc, ((0, pad_b), (0, 0)))
        ctx = jnp.pad(ctx, ((0, pad_b), (0, 0)))
        smp = jnp.pad(smp, ((0, pad_b), (0, 0)))

    para3 = paragraph_matrix.reshape(n_docs, 1, D)
    word3 = word_matrix.reshape(n_words, 1, D)
    outT3 = jnp.swapaxes(outputs, 0, 1).reshape(n_words, 1, D)

    res = pl.pallas_call(
        _dm_kernel,
        grid=(B_pad // TB,),
        in_specs=[
            pl.BlockSpec((TB, 1), lambda i: (i, 0),
                         memory_space=pltpu.SMEM),
            pl.BlockSpec((TB, C), lambda i: (i, 0),
                         memory_space=pltpu.SMEM),
            pl.BlockSpec((TB, S), lambda i: (i, 0),
                         memory_space=pltpu.SMEM),
            pl.BlockSpec(memory_space=pl.ANY),
            pl.BlockSpec((n_words, 1, D), lambda i: (0, 0, 0)),
            pl.BlockSpec((n_words, 1, D), lambda i: (0, 0, 0)),
        ],
        out_specs=pl.BlockSpec((TB, S), lambda i: (i, 0)),
        out_shape=jax.ShapeDtypeStruct((B_pad, S), jnp.float32),
        scratch_shapes=[pltpu.VMEM((TB, 1, D), jnp.float32),
                        pltpu.SemaphoreType.DMA],
        compiler_params=pltpu.CompilerParams(
            dimension_semantics=("parallel",),
            vmem_limit_bytes=64 * 1024 * 1024),
    )(doc, ctx, smp, para3, word3, outT3)

    return jnp.squeeze(res[:B])


# speedup vs baseline: 7.2187x; 1.0108x over previous
import jax
import jax.numpy as jnp
from jax.experimental import pallas as pl
from jax.experimental.pallas import tpu as pltpu


def _dm_kernel(doc_s, ctx_v, smp_s,
               para_hbm,
               word2, outT3,
               o_ref,
               pbuf,
               psem):
    TB, S = o_ref.shape
    C = ctx_v.shape[1]
    NW = word2.shape[0]

    lane = jax.lax.broadcasted_iota(jnp.int32, (1, S), 1)

    for k in range(TB):
        pltpu.make_async_copy(para_hbm.at[doc_s[k, 0]],
                              pbuf.at[k], psem).start()

    def _tree_sum(vals):
        while len(vals) > 1:
            nxt = [a + b for a, b in zip(vals[::2], vals[1::2])]
            if len(vals) % 2:
                nxt.append(vals[-1])
            vals = nxt
        return vals[0]

    ids = ctx_v[...]
    iota_nw = jax.lax.broadcasted_iota(jnp.int32, (TB, NW), 1)
    cnt = _tree_sum([(ids[:, c:c + 1] == iota_nw).astype(jnp.float32)
                     for c in range(C)])
    csum = jnp.dot(cnt, word2[...],
                   preferred_element_type=jnp.float32)

    pltpu.make_async_copy(pbuf, pbuf, psem).wait()

    rows = []
    for r in range(TB):
        acc = csum[r:r + 1, :] + pbuf[r]
        cols = []
        for s in range(S):
            g = outT3[smp_s[r, s]]
            v = jnp.sum(acc * g, axis=-1, keepdims=True)
            cols.append(jnp.where(lane == s, v, 0.0))
        rows.append(_tree_sum(cols))
    o_ref[...] = jnp.concatenate(rows, axis=0)


def kernel(doc_ids, context_ids, sample_ids, paragraph_matrix, word_matrix,
           outputs):
    B, C = context_ids.shape
    S = sample_ids.shape[1]
    n_docs, D = paragraph_matrix.shape
    n_words = word_matrix.shape[0]

    TB = 128 if B % 128 == 0 else 8
    B_pad = ((B + TB - 1) // TB) * TB

    pad_b = B_pad - B
    doc = doc_ids.astype(jnp.int32).reshape(B, 1)
    ctx = context_ids.astype(jnp.int32)
    smp = sample_ids.astype(jnp.int32)
    if pad_b:
        doc = jnp.pad(doc, ((0, pad_b), (0, 0)))
        ctx = jnp.pad(ctx, ((0, pad_b), (0, 0)))
        smp = jnp.pad(smp, ((0, pad_b), (0, 0)))

    para3 = paragraph_matrix.reshape(n_docs, 1, D)
    outT3 = jnp.swapaxes(outputs, 0, 1).reshape(n_words, 1, D)

    res = pl.pallas_call(
        _dm_kernel,
        grid=(B_pad // TB,),
        in_specs=[
            pl.BlockSpec((TB, 1), lambda i: (i, 0),
                         memory_space=pltpu.SMEM),
            pl.BlockSpec((TB, C), lambda i: (i, 0)),
            pl.BlockSpec((TB, S), lambda i: (i, 0),
                         memory_space=pltpu.SMEM),
            pl.BlockSpec(memory_space=pl.ANY),
            pl.BlockSpec((n_words, D), lambda i: (0, 0)),
            pl.BlockSpec((n_words, 1, D), lambda i: (0, 0, 0)),
        ],
        out_specs=pl.BlockSpec((TB, S), lambda i: (i, 0)),
        out_shape=jax.ShapeDtypeStruct((B_pad, S), jnp.float32),
        scratch_shapes=[pltpu.VMEM((TB, 1, D), jnp.float32),
                        pltpu.SemaphoreType.DMA],
        compiler_params=pltpu.CompilerParams(
            dimension_semantics=("parallel",),
            vmem_limit_bytes=64 * 1024 * 1024),
    )(doc, ctx, smp, para3, word_matrix, outT3)

    return jnp.squeeze(res[:B])
